# baseline (speedup 1.0000x reference)
import jax
import jax.numpy as jnp
from jax.experimental import pallas as pl
from jax.experimental.pallas import tpu as pltpu

_CH = 4096
_NB = 1


def _body(idx_ref, arr_any, elem_ref, out_any, bufs, insems, outsems):
    M = arr_any.shape[0]
    nch = M // _CH
    idx = idx_ref[0]
    owner = idx // _CH
    local = idx - owner * _CH

    def in_copy(k):
        b = k % _NB
        return pltpu.make_async_copy(
            arr_any.at[pl.ds(k * _CH, _CH)], bufs.at[b], insems.at[b])

    def out_copy(k):
        b = k % _NB
        return pltpu.make_async_copy(
            bufs.at[b], out_any.at[pl.ds(k * _CH, _CH)], outsems.at[b])

    for k in range(_NB):
        in_copy(k).start()
    for k in range(nch):
        b = k % _NB
        in_copy(k).wait()

        @pl.when(owner == k)
        def _(b=b):
            pltpu.make_async_copy(
                elem_ref, bufs.at[b, pl.ds(local, 1)], insems.at[b]).start()
            pltpu.make_async_copy(
                elem_ref, bufs.at[b, pl.ds(local, 1)], insems.at[b]).wait()

        out_copy(k).start()
        if k + _NB < nch:
            out_copy(k).wait()
            in_copy(k + _NB).start()
    for k in range(nch - _NB, nch):
        if k >= 0:
            out_copy(k).wait()


def kernel(arr, index, element):
    M, N = arr.shape
    idx = jnp.asarray(index, jnp.int32).reshape((1,))
    elem2d = element.reshape((1, N))
    return pl.pallas_call(
        _body,
        grid_spec=pltpu.PrefetchScalarGridSpec(
            num_scalar_prefetch=1,
            grid=(1,),
            in_specs=[
                pl.BlockSpec(memory_space=pl.ANY),
                pl.BlockSpec((1, N), lambda i, idx_ref: (0, 0)),
            ],
            out_specs=pl.BlockSpec(memory_space=pl.ANY),
            scratch_shapes=[
                pltpu.VMEM((_NB, _CH, N), jnp.float32),
                pltpu.SemaphoreType.DMA((_NB,)),
                pltpu.SemaphoreType.DMA((_NB,)),
            ],
        ),
        out_shape=jax.ShapeDtypeStruct((M, N), arr.dtype),
    )(idx, arr, elem2d)


# speedup vs baseline: 1.0632x; 1.0632x over previous
import jax
import jax.numpy as jnp
from jax.experimental import pallas as pl
from jax.experimental.pallas import tpu as pltpu

_CH = 2048
_NB = 2


def _body(idx_ref, arr_any, elem_ref, out_any, bufs, insems, outsems):
    M = arr_any.shape[0]
    nch = M // _CH
    idx = idx_ref[0]
    owner = idx // _CH
    local = idx - owner * _CH

    def in_copy(k):
        b = k % _NB
        return pltpu.make_async_copy(
            arr_any.at[pl.ds(k * _CH, _CH)], bufs.at[b], insems.at[b])

    def out_copy(k):
        b = k % _NB
        return pltpu.make_async_copy(
            bufs.at[b], out_any.at[pl.ds(k * _CH, _CH)], outsems.at[b])

    for k in range(_NB):
        in_copy(k).start()
    for k in range(nch):
        b = k % _NB
        in_copy(k).wait()

        @pl.when(owner == k)
        def _(b=b):
            pltpu.make_async_copy(
                elem_ref, bufs.at[b, pl.ds(local, 1)], insems.at[b]).start()
            pltpu.make_async_copy(
                elem_ref, bufs.at[b, pl.ds(local, 1)], insems.at[b]).wait()

        out_copy(k).start()
        if k + _NB < nch:
            out_copy(k).wait()
            in_copy(k + _NB).start()
    for k in range(nch - _NB, nch):
        if k >= 0:
            out_copy(k).wait()


def kernel(arr, index, element):
    M, N = arr.shape
    idx = jnp.asarray(index, jnp.int32).reshape((1,))
    elem2d = element.reshape((1, N))
    return pl.pallas_call(
        _body,
        grid_spec=pltpu.PrefetchScalarGridSpec(
            num_scalar_prefetch=1,
            grid=(1,),
            in_specs=[
                pl.BlockSpec(memory_space=pl.ANY),
                pl.BlockSpec((1, N), lambda i, idx_ref: (0, 0)),
            ],
            out_specs=pl.BlockSpec(memory_space=pl.ANY),
            scratch_shapes=[
                pltpu.VMEM((_NB, _CH, N), jnp.float32),
                pltpu.SemaphoreType.DMA((_NB,)),
                pltpu.SemaphoreType.DMA((_NB,)),
            ],
        ),
        out_shape=jax.ShapeDtypeStruct((M, N), arr.dtype),
    )(idx, arr, elem2d)


# speedup vs baseline: 1.0910x; 1.0261x over previous
import jax
import jax.numpy as jnp
from jax.experimental import pallas as pl
from jax.experimental.pallas import tpu as pltpu

_CHUNKS = (256, 1792, 1792, 256)


def _body(idx_ref, arr_any, elem_ref, out_any, *rest):
    n = len(_CHUNKS)
    bufs = rest[:n]
    insems = rest[n]
    outsems = rest[n + 1]
    idx = idx_ref[0]

    starts = []
    s = 0
    for ch in _CHUNKS:
        starts.append(s)
        s += ch

    def in_copy(k):
        return pltpu.make_async_copy(
            arr_any.at[pl.ds(starts[k], _CHUNKS[k])], bufs[k], insems.at[k])

    def out_copy(k):
        return pltpu.make_async_copy(
            bufs[k], out_any.at[pl.ds(starts[k], _CHUNKS[k])], outsems.at[k])

    for k in range(n):
        in_copy(k).start()
    for k in range(n):
        in_copy(k).wait()

        @pl.when((idx >= starts[k]) & (idx < starts[k] + _CHUNKS[k]))
        def _(k=k):
            patch = pltpu.make_async_copy(
                elem_ref, bufs[k].at[pl.ds(idx - starts[k], 1)], insems.at[k])
            patch.start()
            patch.wait()

        out_copy(k).start()
    for k in range(n):
        out_copy(k).wait()


def kernel(arr, index, element):
    M, N = arr.shape
    idx = jnp.asarray(index, jnp.int32).reshape((1,))
    elem2d = element.reshape((1, N))
    return pl.pallas_call(
        _body,
        grid_spec=pltpu.PrefetchScalarGridSpec(
            num_scalar_prefetch=1,
            grid=(1,),
            in_specs=[
                pl.BlockSpec(memory_space=pl.ANY),
                pl.BlockSpec((1, N), lambda i, idx_ref: (0, 0)),
            ],
            out_specs=pl.BlockSpec(memory_space=pl.ANY),
            scratch_shapes=(
                [pltpu.VMEM((ch, N), jnp.float32) for ch in _CHUNKS]
                + [pltpu.SemaphoreType.DMA((len(_CHUNKS),)),
                   pltpu.SemaphoreType.DMA((len(_CHUNKS),))]
            ),
        ),
        out_shape=jax.ShapeDtypeStruct((M, N), arr.dtype),
    )(idx, arr, elem2d)


# speedup vs baseline: 1.1303x; 1.0360x over previous
import jax
import jax.numpy as jnp
from jax.experimental import pallas as pl
from jax.experimental.pallas import tpu as pltpu

_CHUNKS = (128, 512, 1408, 1408, 512, 128)


def _body(idx_ref, arr_any, elem_ref, out_any, *rest):
    n = len(_CHUNKS)
    bufs = rest[:n]
    insems = rest[n]
    outsems = rest[n + 1]
    idx = idx_ref[0]

    starts = []
    s = 0
    for ch in _CHUNKS:
        starts.append(s)
        s += ch

    def in_copy(k):
        return pltpu.make_async_copy(
            arr_any.at[pl.ds(starts[k], _CHUNKS[k])], bufs[k], insems.at[k])

    def out_copy(k):
        return pltpu.make_async_copy(
            bufs[k], out_any.at[pl.ds(starts[k], _CHUNKS[k])], outsems.at[k])

    for k in range(n):
        in_copy(k).start()
    for k in range(n):
        in_copy(k).wait()

        @pl.when((idx >= starts[k]) & (idx < starts[k] + _CHUNKS[k]))
        def _(k=k):
            patch = pltpu.make_async_copy(
                elem_ref, bufs[k].at[pl.ds(idx - starts[k], 1)], insems.at[k])
            patch.start()
            patch.wait()

        out_copy(k).start()
    for k in range(n):
        out_copy(k).wait()


def kernel(arr, index, element):
    M, N = arr.shape
    idx = jnp.asarray(index, jnp.int32).reshape((1,))
    elem2d = element.reshape((1, N))
    return pl.pallas_call(
        _body,
        grid_spec=pltpu.PrefetchScalarGridSpec(
            num_scalar_prefetch=1,
            grid=(1,),
            in_specs=[
                pl.BlockSpec(memory_space=pl.ANY),
                pl.BlockSpec((1, N), lambda i, idx_ref: (0, 0)),
            ],
            out_specs=pl.BlockSpec(memory_space=pl.ANY),
            scratch_shapes=(
                [pltpu.VMEM((ch, N), jnp.float32) for ch in _CHUNKS]
                + [pltpu.SemaphoreType.DMA((len(_CHUNKS),)),
                   pltpu.SemaphoreType.DMA((len(_CHUNKS),))]
            ),
        ),
        out_shape=jax.ShapeDtypeStruct((M, N), arr.dtype),
    )(idx, arr, elem2d)
